# baseline (speedup 1.0000x reference)
import functools

import jax
import jax.numpy as jnp
from jax.experimental import pallas as pl
from jax.experimental.pallas import tpu as pltpu

M = 1024
K = 16
N = 100000
TILE_N = 2048
NSLOTS = 4
NT = (N + TILE_N - 1) // TILE_N
LAST_W = N - (NT - 1) * TILE_N


def _mm_kernel(x_ref, em_ref, o_ref, acc_ref, tail_ref, sem_ref, tail_sem):
    i = pl.program_id(0)
    slot = jax.lax.rem(i, NSLOTS)

    for s in range(NSLOTS):
        @pl.when(jnp.logical_and(slot == s,
                                 jnp.logical_and(i >= NSLOTS, i < NT - 1)))
        def _wait_prev(s=s):
            pltpu.make_async_copy(
                acc_ref.at[s],
                o_ref.at[:, pl.ds((i - NSLOTS) * TILE_N, TILE_N)],
                sem_ref.at[s],
            ).wait()

        @pl.when(jnp.logical_and(slot == s, i < NT - 1))
        def _store_full(s=s):
            acc_ref[s] = jax.lax.dot_general(
                x_ref[...], em_ref[...],
                dimension_numbers=(((1,), (1,)), ((), ())),
                preferred_element_type=jnp.float32,
            )
            pltpu.make_async_copy(
                acc_ref.at[s],
                o_ref.at[:, pl.ds(i * TILE_N, TILE_N)],
                sem_ref.at[s],
            ).start()

    @pl.when(i == NT - 1)
    def _store_last_and_drain():
        tail_ref[...] = jax.lax.dot_general(
            x_ref[...], em_ref[:LAST_W, :],
            dimension_numbers=(((1,), (1,)), ((), ())),
            preferred_element_type=jnp.float32,
        )
        last = pltpu.make_async_copy(
            tail_ref,
            o_ref.at[:, pl.ds((NT - 1) * TILE_N, LAST_W)],
            tail_sem,
        )
        last.start()
        for back in range(1, NSLOTS + 1):
            j = NT - 1 - back
            if j >= 0:
                pltpu.make_async_copy(
                    acc_ref.at[j % NSLOTS],
                    o_ref.at[:, pl.ds(j * TILE_N, TILE_N)],
                    sem_ref.at[j % NSLOTS],
                ).wait()
        last.wait()


@functools.partial(jax.jit, static_argnames=())
def kernel(inputs, targets, em):
    del targets
    out = pl.pallas_call(
        _mm_kernel,
        grid=(NT,),
        in_specs=[
            pl.BlockSpec((M, K), lambda i: (0, 0)),
            pl.BlockSpec((TILE_N, K), lambda i: (i, 0)),
        ],
        out_specs=pl.BlockSpec(memory_space=pl.ANY),
        out_shape=jax.ShapeDtypeStruct((M, N), jnp.float32),
        scratch_shapes=[
            pltpu.VMEM((NSLOTS, M, TILE_N), jnp.float32),
            pltpu.VMEM((M, LAST_W), jnp.float32),
            pltpu.SemaphoreType.DMA((NSLOTS,)),
            pltpu.SemaphoreType.DMA,
        ],
        compiler_params=pltpu.CompilerParams(
            dimension_semantics=("arbitrary",),
        ),
    )(inputs, em)
    return out


# speedup vs baseline: 3.0938x; 3.0938x over previous
import functools

import jax
import jax.numpy as jnp
from jax.experimental import pallas as pl
from jax.experimental.pallas import tpu as pltpu

TILE_R = 3136


def _mm_kernel(em_ref, x_ref, o_ref):
    o_ref[...] = jax.lax.dot_general(
        em_ref[...], x_ref[...],
        dimension_numbers=(((1,), (1,)), ((), ())),
        preferred_element_type=jnp.float32,
    )


@functools.partial(jax.jit, static_argnames=())
def kernel(inputs, targets, em):
    del targets
    m, k = inputs.shape
    n = em.shape[0]
    out_t = pl.pallas_call(
        _mm_kernel,
        grid=(pl.cdiv(n, TILE_R),),
        in_specs=[
            pl.BlockSpec((TILE_R, k), lambda i: (i, 0)),
            pl.BlockSpec((m, k), lambda i: (0, 0)),
        ],
        out_specs=pl.BlockSpec((TILE_R, m), lambda i: (i, 0)),
        out_shape=jax.ShapeDtypeStruct((n, m), jnp.float32),
        compiler_params=pltpu.CompilerParams(
            dimension_semantics=("arbitrary",),
        ),
    )(em, inputs)
    return out_t.T
